# baseline (speedup 1.0000x reference)
import functools

import jax
import jax.numpy as jnp
from jax import lax
from jax.experimental import pallas as pl
from jax.experimental.pallas import tpu as pltpu
from jax.experimental.pallas import tpu_sc as plsc

VOCAB = 1000000
EMBED = 32
B = 4096
S = 200

NC = 2
NS = 16
NW = NC * NS
BL = 128
SB = S // 8
EB = EMBED // 8
NBUF = 8

_mesh = plsc.VectorSubcoreMesh(core_axis_name="c", subcore_axis_name="s")


@functools.partial(
    pl.kernel,
    mesh=_mesh,
    out_type=jax.ShapeDtypeStruct((S, EB, NW, 8, BL), jnp.float32),
    scratch_types=[
        pltpu.VMEM((SB, 8, BL), jnp.int32),
        pltpu.VMEM((NBUF, BL, EMBED), jnp.float32),
        pltpu.VMEM((NBUF, EB, 8, BL), jnp.float32),
        pltpu.SemaphoreType.DMA((NBUF,)),
        pltpu.SemaphoreType.DMA((NBUF,)),
    ],
    compiler_params=pltpu.CompilerParams(
        use_tc_tiling_on_sc=False, needs_layout_passes=False
    ),
)
def _emb_lookup(x4_hbm, table_hbm, out_hbm, idx_v, rows_v, tbuf, gsem, ssem):
    wid = lax.axis_index("s") * NC + lax.axis_index("c")

    pltpu.sync_copy(x4_hbm.at[:, wid], idx_v)

    iota16 = lax.iota(jnp.int32, 16)

    def gather_start(s, b):
        pltpu.async_copy(
            table_hbm.at[idx_v.at[s // 8, s % 8]], rows_v.at[b], gsem.at[b]
        )

    def gather_wait(s, b):
        pltpu.make_async_copy(
            table_hbm.at[idx_v.at[s // 8, s % 8]], rows_v.at[b], gsem.at[b]
        ).wait()

    def store_start(s, b):
        pltpu.async_copy(
            tbuf.at[b], out_hbm.at[s, pl.ds(0, EB), wid], ssem.at[b]
        )

    def store_wait(s, b):
        pltpu.make_async_copy(
            tbuf.at[b], out_hbm.at[s, pl.ds(0, EB), wid], ssem.at[b]
        ).wait()

    def transpose(b):
        src = rows_v.at[b]
        for e in range(EMBED):
            col = jnp.full((16,), e, jnp.int32)
            for g in range(BL // 16):
                v = plsc.load_gather(src, [iota16 + 16 * g, col])
                tbuf[b, e // 8, e % 8, pl.ds(16 * g, 16)] = v

    for b in range(NBUF):
        gather_start(b, b)

    def unit(s, carry):
        b = s % NBUF
        gather_wait(s, b)

        @pl.when(s >= NBUF)
        def _():
            store_wait(s - NBUF, b)

        transpose(b)
        store_start(s, b)

        @pl.when(s + NBUF < S)
        def _():
            gather_start(s + NBUF, b)

        return carry

    lax.fori_loop(0, S, unit, 0)

    for k in range(NBUF):
        s = S - NBUF + k
        store_wait(s, s % NBUF)


def kernel(x, table):
    x4 = x.T.reshape(SB, 8, NW, BL).transpose(0, 2, 1, 3)
    out5 = _emb_lookup(x4, table)
    return out5.transpose(2, 4, 0, 1, 3).reshape(B, S, EMBED)


# speedup vs baseline: 2.6052x; 2.6052x over previous
import functools

import jax
import jax.numpy as jnp
from jax import lax
from jax.experimental import pallas as pl
from jax.experimental.pallas import tpu as pltpu
from jax.experimental.pallas import tpu_sc as plsc

VOCAB = 1000000
EMBED = 32
B = 4096
S = 200

NC = 2
NS = 16
NW = NC * NS
BL = 128
SB = S // 8
EB = EMBED // 8
NBUF = 8

_mesh = plsc.VectorSubcoreMesh(core_axis_name="c", subcore_axis_name="s")


VC = VOCAB // 128
T1U = VC // NW
NB1 = 4


@functools.partial(
    pl.kernel,
    mesh=_mesh,
    out_type=jax.ShapeDtypeStruct((VOCAB // 4, 128), jnp.float32),
    scratch_types=[
        pltpu.VMEM((NB1, 32, 128), jnp.float32),
        pltpu.VMEM((NB1, 32, 128), jnp.float32),
        pltpu.SemaphoreType.DMA((NB1,)),
        pltpu.SemaphoreType.DMA((NB1,)),
    ],
    compiler_params=pltpu.CompilerParams(
        use_tc_tiling_on_sc=True, needs_layout_passes=False
    ),
)
def _table_relayout(tT_hbm, tail_hbm, out_hbm, src_v, dst_v, lsem, ssem):
    wid = lax.axis_index("s") * NC + lax.axis_index("c")
    iota16 = lax.iota(jnp.int32, 16)
    l32 = iota16 * 32

    def load_start(u, b):
        pltpu.async_copy(
            tT_hbm.at[:, pl.ds(u * 128, 128)], src_v.at[b], lsem.at[b]
        )

    def load_wait(u, b):
        pltpu.make_async_copy(
            tT_hbm.at[:, pl.ds(u * 128, 128)], src_v.at[b], lsem.at[b]
        ).wait()

    def store_start(u, b):
        pltpu.async_copy(
            dst_v.at[b], out_hbm.at[pl.ds(u * 32, 32)], ssem.at[b]
        )

    def store_wait(u, b):
        pltpu.make_async_copy(
            dst_v.at[b], out_hbm.at[pl.ds(u * 32, 32)], ssem.at[b]
        ).wait()

    def transpose(b, nbb):
        src = src_v.at[b]
        dst = dst_v.at[b]

        def diag(j, carry):
            for i in range(8):
                t = (iota16 + (j * 8 + i)) & 15
                l32pt = l32 + t
                for eb in range(2):
                    ev = t + 16 * eb
                    for bb in range(nbb):
                        blv = iota16 + 16 * bb
                        v = plsc.load_gather(src, [ev, blv])
                        off = l32pt + (512 * bb + 16 * eb)
                        plsc.store_scatter(dst, [off >> 7, off & 127], v)
            return carry

        lax.fori_loop(0, 2, diag, 0)

    for b in range(NB1):
        load_start(wid + 32 * b, b)

    def unit(k, carry):
        b = k % NB1
        u = wid + 32 * k
        load_wait(u, b)

        @pl.when(k >= NB1)
        def _():
            store_wait(wid + 32 * (k - NB1), b)

        transpose(b, 8)
        store_start(u, b)

        @pl.when(k + NB1 < T1U)
        def _():
            load_start(wid + 32 * (k + NB1), b)

        return carry

    lax.fori_loop(0, T1U, unit, 0)

    for k in range(T1U - NB1, T1U):
        store_wait(wid + 32 * k, k % NB1)

    @pl.when(wid < 4)
    def _():
        u = VC - 4 + wid
        pltpu.sync_copy(tT_hbm.at[:, pl.ds(u * 128, 128)], src_v.at[0])
        transpose(0, 8)
        pltpu.sync_copy(dst_v.at[0], out_hbm.at[pl.ds(u * 32, 32)])

    @pl.when(wid == 4)
    def _():
        pltpu.sync_copy(tail_hbm, dst_v.at[0, pl.ds(0, 16)])
        pltpu.sync_copy(
            dst_v.at[0, pl.ds(0, 16)], out_hbm.at[pl.ds(VC * 32, 16)]
        )


@functools.partial(
    pl.kernel,
    mesh=_mesh,
    out_type=jax.ShapeDtypeStruct((S, EB, NW, 8, BL), jnp.float32),
    scratch_types=[
        pltpu.VMEM((SB, 8, BL), jnp.int32),
        pltpu.VMEM((NBUF, BL, EMBED), jnp.float32),
        pltpu.VMEM((NBUF, EB, 8, BL), jnp.float32),
        pltpu.SemaphoreType.DMA((NBUF,)),
        pltpu.SemaphoreType.DMA((NBUF,)),
    ],
    compiler_params=pltpu.CompilerParams(
        use_tc_tiling_on_sc=False, needs_layout_passes=False
    ),
)
def _emb_lookup(x4_hbm, table_hbm, out_hbm, idx_v, rows_v, tbuf, gsem, ssem):
    wid = lax.axis_index("s") * NC + lax.axis_index("c")

    pltpu.sync_copy(x4_hbm.at[:, wid], idx_v)

    iota16 = lax.iota(jnp.int32, 16)

    def gather_start(s, b):
        pltpu.async_copy(
            table_hbm.at[idx_v.at[s // 8, s % 8]], rows_v.at[b], gsem.at[b]
        )

    def gather_wait(s, b):
        pltpu.make_async_copy(
            table_hbm.at[idx_v.at[s // 8, s % 8]], rows_v.at[b], gsem.at[b]
        ).wait()

    def store_start(s, b):
        pltpu.async_copy(
            tbuf.at[b], out_hbm.at[s, pl.ds(0, EB), wid], ssem.at[b]
        )

    def store_wait(s, b):
        pltpu.make_async_copy(
            tbuf.at[b], out_hbm.at[s, pl.ds(0, EB), wid], ssem.at[b]
        ).wait()

    def transpose(b):
        src = rows_v.at[b]
        dst = tbuf.at[b]

        def diag(j, carry):
            for i in range(8):
                rot = (iota16 + (j * 8 + i)) & 15
                for eb in range(EMBED // 16):
                    ec = iota16 + 16 * eb
                    e0 = ec >> 3
                    es = ec & 7
                    for bb in range(BL // 16):
                        blv = rot + 16 * bb
                        v = plsc.load_gather(src, [blv, ec])
                        plsc.store_scatter(dst, [e0, es, blv], v)
            return carry

        lax.fori_loop(0, 2, diag, 0)

    for b in range(NBUF):
        gather_start(b, b)

    def unit(s, carry):
        b = s % NBUF
        gather_wait(s, b)

        @pl.when(s >= NBUF)
        def _():
            store_wait(s - NBUF, b)

        transpose(b)
        store_start(s, b)

        @pl.when(s + NBUF < S)
        def _():
            gather_start(s + NBUF, b)

        return carry

    lax.fori_loop(0, S, unit, 0)

    for k in range(NBUF):
        s = S - NBUF + k
        store_wait(s, s % NBUF)


def kernel(x, table):
    x4 = x.T.reshape(SB, 8, NW, BL).transpose(0, 2, 1, 3)
    tail4 = table[VC * 128 :].reshape(16, 128)
    table_lin = _table_relayout(table.T, tail4).reshape(VOCAB, EMBED)
    out5 = _emb_lookup(x4, table_lin)
    return out5.transpose(2, 4, 0, 1, 3).reshape(B, S, EMBED)
